# Initial kernel scaffold; baseline (speedup 1.0000x reference)
import jax
import jax.numpy as jnp
from jax.experimental import pallas as pl


def kernel(pred_image, sketch, matte):
    raise NotImplementedError("write your pallas kernel here")


# speedup vs baseline: 44.4442x; 44.4442x over previous
import numpy as np
import jax
import jax.numpy as jnp
from jax import lax
from jax.experimental import pallas as pl
from jax.experimental.pallas import tpu as pltpu

_LOW_T = 0.1
_HIGH_T = 0.2
_EPS = 1e-6
_TAN22 = 0.41421356237309503

_xs = (np.arange(5) - 2).astype(np.float32)
_g = np.exp(-(_xs * _xs) / np.float32(2.0)).astype(np.float32)
_g = (_g / _g.sum()).astype(np.float32)
_G0, _G1, _G2 = float(_g[0]), float(_g[1]), float(_g[2])


def _canny_loss_kernel(pred_ref, sk_ref, mt_ref, out_ref):
    H, W = mt_ref.shape[-2], mt_ref.shape[-1]
    f32 = jnp.float32

    mt = mt_ref[0, 0]
    gray = (pred_ref[0, 0] * mt + pred_ref[0, 1] * mt + pred_ref[0, 2] * mt) / 3.0

    xp = jnp.concatenate(
        [gray[2:3], gray[1:2], gray, gray[H - 2:H - 1], gray[H - 3:H - 2]], axis=0)
    v = (_G0 * (xp[0:H] + xp[4:H + 4])
         + _G1 * (xp[1:H + 1] + xp[3:H + 3])
         + _G2 * xp[2:H + 2])
    hp = jnp.concatenate(
        [v[:, 2:3], v[:, 1:2], v, v[:, W - 2:W - 1], v[:, W - 3:W - 2]], axis=1)
    blur = (_G0 * (hp[:, 0:W] + hp[:, 4:W + 4])
            + _G1 * (hp[:, 1:W + 1] + hp[:, 3:W + 3])
            + _G2 * hp[:, 2:W + 2])

    bp = jnp.concatenate([blur[0:1], blur, blur[H - 1:H]], axis=0)
    bp = jnp.concatenate([bp[:, 0:1], bp, bp[:, W - 1:W]], axis=1)
    a00 = bp[0:H, 0:W]; a01 = bp[0:H, 1:W + 1]; a02 = bp[0:H, 2:W + 2]
    a10 = bp[1:H + 1, 0:W]; a12 = bp[1:H + 1, 2:W + 2]
    a20 = bp[2:H + 2, 0:W]; a21 = bp[2:H + 2, 1:W + 1]; a22 = bp[2:H + 2, 2:W + 2]
    gx = (a02 - a00 + 2.0 * (a12 - a10) + a22 - a20) * 0.125
    gy = (a20 - a00 + 2.0 * (a21 - a01) + a22 - a02) * 0.125

    mag = jnp.sqrt(gx * gx + gy * gy + _EPS)

    zr = jnp.zeros((1, W), f32)
    zc = jnp.zeros((H + 2, 1), f32)
    mp = jnp.concatenate([zr, mag, zr], axis=0)
    mp = jnp.concatenate([zc, mp, zc], axis=1)
    nE = mp[1:H + 1, 2:W + 2]; nW = mp[1:H + 1, 0:W]
    nN = mp[0:H, 1:W + 1];     nS = mp[2:H + 2, 1:W + 1]
    nNE = mp[0:H, 2:W + 2];    nSW = mp[2:H + 2, 0:W]
    nNW = mp[0:H, 0:W];        nSE = mp[2:H + 2, 2:W + 2]

    ax = jnp.abs(gx)
    ay = jnp.abs(gy)
    is_h = ay <= _TAN22 * ax
    is_v = jnp.logical_and(jnp.logical_not(is_h), ax <= _TAN22 * ay)
    diag_main = (gx * gy) > 0.0
    na = jnp.where(is_h, nE, jnp.where(is_v, nN, jnp.where(diag_main, nNE, nNW)))
    nb = jnp.where(is_h, nW, jnp.where(is_v, nS, jnp.where(diag_main, nSW, nSE)))
    is_max = jnp.logical_and(mag > na, mag > nb)

    strong_c = jnp.logical_and(is_max, mag > _HIGH_T)
    weak_c = jnp.logical_and(is_max, jnp.logical_and(mag > _LOW_T, mag <= _HIGH_T))
    s = jnp.where(strong_c, 1.0, 0.0).astype(f32)
    w = jnp.where(weak_c, 1.0, 0.0).astype(f32)

    zc2 = jnp.zeros((H, 1), f32)

    def dilate(sa):
        rp = jnp.concatenate([zr, sa, zr], axis=0)
        rm = jnp.maximum(jnp.maximum(rp[0:H], rp[1:H + 1]), rp[2:H + 2])
        cpad = jnp.concatenate([zc2, rm, zc2], axis=1)
        return jnp.maximum(jnp.maximum(cpad[:, 0:W], cpad[:, 1:W + 1]),
                           cpad[:, 2:W + 2])

    def cond_fn(c):
        return c[1]

    def body_fn(c):
        s_cur, _ = c
        s_new = jnp.maximum(s_cur, w * dilate(s_cur))
        s_new = jnp.maximum(s_new, w * dilate(s_new))
        changed = jnp.any(s_new > s_cur)
        return (s_new, changed)

    s_fix, _ = lax.while_loop(cond_fn, body_fn, (s, jnp.bool_(True)))

    edges = s_fix + 0.5 * w * (1.0 - s_fix)

    tgt = jnp.clip(sk_ref[0, 0] * mt, 0.0, 1.0)
    diff = edges - tgt
    out_ref[0] = jnp.sum(diff * diff, axis=0, keepdims=True)


def kernel(pred_image, sketch, matte):
    B, C, H, W = pred_image.shape
    partial = pl.pallas_call(
        _canny_loss_kernel,
        grid=(B,),
        in_specs=[
            pl.BlockSpec((1, C, H, W), lambda b: (b, 0, 0, 0)),
            pl.BlockSpec((1, 1, H, W), lambda b: (b, 0, 0, 0)),
            pl.BlockSpec((1, 1, H, W), lambda b: (b, 0, 0, 0)),
        ],
        out_specs=pl.BlockSpec((1, 1, W), lambda b: (b, 0, 0)),
        out_shape=jax.ShapeDtypeStruct((B, 1, W), jnp.float32),
        compiler_params=pltpu.CompilerParams(
            dimension_semantics=("parallel",),
        ),
    )(pred_image, sketch, matte)
    return jnp.sum(partial) / (B * H * W)


# speedup vs baseline: 107.3412x; 2.4152x over previous
import numpy as np
import jax
import jax.numpy as jnp
from jax import lax
from jax.experimental import pallas as pl
from jax.experimental.pallas import tpu as pltpu

_LOW_T2 = float(np.float32(0.1) * np.float32(0.1))
_HIGH_T2 = float(np.float32(0.2) * np.float32(0.2))
_EPS = 1e-6
_TAN22 = 0.41421356237309503


def _gauss5():
    xs = (np.arange(5) - 2).astype(np.float32)
    g = np.exp(-(xs * xs) / np.float32(2.0)).astype(np.float32)
    return (g / g.sum()).astype(np.float64)


def _stencil_mats(n):
    g = _gauss5()
    refl = lambda i: min(max(-i if i < 0 else (2 * n - 2 - i if i >= n else i), 0), n - 1)
    rep = lambda i: min(max(i, 0), n - 1)
    B = np.zeros((n, n))
    A = np.zeros((n, n))
    D = np.zeros((n, n))
    for i in range(n):
        for k in range(-2, 3):
            B[i, refl(i + k)] += g[k + 2]
        A[i, rep(i - 1)] += 0.125
        A[i, i] += 0.25
        A[i, rep(i + 1)] += 0.125
        D[i, rep(i + 1)] += 0.125
        D[i, rep(i - 1)] -= 0.125
    return B, A, D


def _canny_loss_kernel(pred_ref, sk_ref, mt_ref, lx_ref, ly_ref, rx_ref,
                       ry_ref, rz_ref, lz_ref, triv_ref, trih_ref, out_ref):
    H, W = mt_ref.shape[-2], mt_ref.shape[-1]
    f32 = jnp.float32
    bf16 = jnp.bfloat16

    mt = mt_ref[0, 0]
    gray = (pred_ref[0, 0] + pred_ref[0, 1] + pred_ref[0, 2]) * mt / 3.0

    gx = jnp.dot(lx_ref[...], jnp.dot(gray, rx_ref[...], preferred_element_type=f32),
                 preferred_element_type=f32)
    gy = jnp.dot(ly_ref[...], jnp.dot(gray, ry_ref[...], preferred_element_type=f32),
                 preferred_element_type=f32)

    mag = gx * gx + gy * gy + _EPS

    zr = jnp.zeros((1, W), f32)

    def up_z(x):
        return jnp.concatenate([zr, x[0:H - 1]], axis=0)

    def dn_z(x):
        return jnp.concatenate([x[1:H], zr], axis=0)

    nE = jnp.dot(mag, rz_ref[...], preferred_element_type=f32)
    nW = jnp.dot(mag, lz_ref[...], preferred_element_type=f32)
    nN = up_z(mag); nS = dn_z(mag)
    nNE = up_z(nE); nSW = dn_z(nW)
    nNW = up_z(nW); nSE = dn_z(nE)

    ax = jnp.abs(gx)
    ay = jnp.abs(gy)
    is_h = ay <= _TAN22 * ax
    is_v = jnp.logical_and(jnp.logical_not(is_h), ax <= _TAN22 * ay)
    diag_main = (gx * gy) > 0.0
    na = jnp.where(is_h, nE, jnp.where(is_v, nN, jnp.where(diag_main, nNE, nNW)))
    nb = jnp.where(is_h, nW, jnp.where(is_v, nS, jnp.where(diag_main, nSW, nSE)))
    is_max = jnp.logical_and(mag > na, mag > nb)

    strong_c = jnp.logical_and(is_max, mag > _HIGH_T2)
    weak_c = jnp.logical_and(is_max, jnp.logical_and(mag > _LOW_T2, mag <= _HIGH_T2))
    s = jnp.where(strong_c, 1.0, 0.0).astype(bf16)
    w = jnp.where(weak_c, 1.0, 0.0).astype(bf16)

    triv = triv_ref[...]
    trih = trih_ref[...]

    def grow(sa):
        t1 = jnp.dot(sa, trih, preferred_element_type=f32)
        t2 = jnp.dot(triv, t1.astype(bf16), preferred_element_type=f32)
        d = jnp.minimum(t2, 1.0).astype(bf16)
        return jnp.maximum(sa, w * d)

    def cond_fn(c):
        return c[1]

    def body_fn(c):
        s_cur, _ = c
        s_new = grow(grow(s_cur))
        changed = jnp.max((s_new - s_cur).astype(f32)) > 0
        return (s_new, changed)

    s_fix, _ = lax.while_loop(cond_fn, body_fn, (s, jnp.bool_(True)))

    edges = (s_fix + 0.5 * w * (1.0 - s_fix)).astype(f32)

    tgt = jnp.clip(sk_ref[0, 0] * mt, 0.0, 1.0)
    diff = edges - tgt
    out_ref[0] = jnp.sum(diff * diff, axis=0, keepdims=True)


def kernel(pred_image, sketch, matte):
    B, C, H, W = pred_image.shape
    Bv, Av, Dv = _stencil_mats(H)
    Bh, Ah, Dh = _stencil_mats(W)
    lx = jnp.asarray(Av @ Bv, jnp.float32)
    ly = jnp.asarray(Dv @ Bv, jnp.float32)
    rx = jnp.asarray(Bh @ Dh, jnp.float32)
    ry = jnp.asarray(Bh @ Ah, jnp.float32)
    rz = jnp.asarray(np.eye(W, k=-1), jnp.float32)
    lz = jnp.asarray(np.eye(W, k=1), jnp.float32)
    iv = jnp.arange(H)
    ih = jnp.arange(W)
    triv = (jnp.abs(iv[:, None] - iv[None, :]) <= 1).astype(jnp.bfloat16)
    trih = (jnp.abs(ih[:, None] - ih[None, :]) <= 1).astype(jnp.bfloat16)
    const2d = lambda shape: pl.BlockSpec(shape, lambda b: (0, 0))
    partial = pl.pallas_call(
        _canny_loss_kernel,
        grid=(B,),
        in_specs=[
            pl.BlockSpec((1, C, H, W), lambda b: (b, 0, 0, 0)),
            pl.BlockSpec((1, 1, H, W), lambda b: (b, 0, 0, 0)),
            pl.BlockSpec((1, 1, H, W), lambda b: (b, 0, 0, 0)),
            const2d((H, H)), const2d((H, H)),
            const2d((W, W)), const2d((W, W)),
            const2d((W, W)), const2d((W, W)),
            const2d((H, H)), const2d((W, W)),
        ],
        out_specs=pl.BlockSpec((1, 1, W), lambda b: (b, 0, 0)),
        out_shape=jax.ShapeDtypeStruct((B, 1, W), jnp.float32),
        compiler_params=pltpu.CompilerParams(
            dimension_semantics=("parallel",),
        ),
    )(pred_image, sketch, matte, lx, ly, rx, ry, rz, lz, triv, trih)
    return jnp.sum(partial) / (B * H * W)
